# baseline (speedup 1.0000x reference)
import functools
import math

import jax
import jax.numpy as jnp
from jax import lax
from jax.experimental import pallas as pl
from jax.experimental.pallas import tpu as pltpu
from jax.experimental.pallas import tpu_sc as plsc

NC = 2
NS = 16
NW = NC * NS
L = 16

SUB = 128
CH = 2
C = SUB * CH


@functools.lru_cache(maxsize=None)
def _make(V, D, B):
    assert B % (NW * C) == 0 and D % L == 0
    bpw = B // NW
    nsub = bpw // SUB
    g_total = nsub // CH
    scale = float(math.sqrt(D))
    mesh = plsc.VectorSubcoreMesh(core_axis_name="c", subcore_axis_name="s")

    @functools.partial(
        pl.kernel,
        mesh=mesh,
        compiler_params=pltpu.CompilerParams(use_tc_tiling_on_sc=False),
        out_type=jax.ShapeDtypeStruct((B, D), jnp.float32),
        scratch_types=[
            pltpu.VMEM((nsub, SUB), jnp.int32),
            pltpu.VMEM((C, D), jnp.float32),
            pltpu.VMEM((C, D), jnp.float32),
            pltpu.VMEM((C, D), jnp.float32),
            pltpu.VMEM((C, D), jnp.float32),
            pltpu.SemaphoreType.DMA,
            pltpu.SemaphoreType.DMA,
            pltpu.SemaphoreType.DMA,
            pltpu.SemaphoreType.DMA,
        ],
    )
    def emb(table_hbm, ids_hbm, out_hbm,
            idx_v, gb0, gb1, sb0, sb1, gsem0, gsem1, ssem0, ssem1):
        wid = lax.axis_index("s") * NC + lax.axis_index("c")
        row0 = wid * bpw
        gbufs, sbufs = (gb0, gb1), (sb0, sb1)
        gsems, ssems = (gsem0, gsem1), (ssem0, ssem1)

        pltpu.sync_copy(ids_hbm.at[wid], idx_v)

        def fire_gather(g, b):
            for j in range(CH):
                pltpu.async_copy(
                    table_hbm.at[idx_v.at[g * CH + j]],
                    gbufs[b].at[pl.ds(j * SUB, SUB)],
                    gsems[b])

        fire_gather(0, 0)
        fire_gather(1, 1)

        def chunk(g, b):
            pltpu.make_async_copy(
                out_hbm.at[pl.ds(0, C)], gbufs[b], gsems[b]).wait()

            @pl.when(g >= 2)
            def _():
                pltpu.make_async_copy(
                    sbufs[b], out_hbm.at[pl.ds(0, C)], ssems[b]).wait()

            def srow(r4, carry):
                for rr in range(4):
                    r = r4 * 4 + rr
                    for j in range(D // L):
                        sbufs[b][r, pl.ds(j * L, L)] = (
                            gbufs[b][r, pl.ds(j * L, L)] * scale)
                return carry
            lax.fori_loop(0, C // 4, srow, 0)

            @pl.when(g + 2 < g_total)
            def _():
                fire_gather(g + 2, b)

            pltpu.async_copy(
                sbufs[b], out_hbm.at[pl.ds(row0 + g * C, C)], ssems[b])

        def outer(i, carry):
            chunk(2 * i, 0)
            chunk(2 * i + 1, 1)
            return carry
        lax.fori_loop(0, g_total // 2, outer, 0)

        for b in range(2):
            pltpu.make_async_copy(
                sbufs[b], out_hbm.at[pl.ds(0, C)], ssems[b]).wait()

    return emb


def kernel(input_ids, table):
    V, D = table.shape
    B = input_ids.size
    bpw = B // NW
    ids3 = input_ids.astype(jnp.int32).reshape(NW, bpw // SUB, SUB)
    out = _make(V, D, B)(table, ids3)
    return out.reshape(*input_ids.shape, D)


# speedup vs baseline: 1.1423x; 1.1423x over previous
import functools
import math

import jax
import jax.numpy as jnp
from jax import lax
from jax.experimental import pallas as pl
from jax.experimental.pallas import tpu as pltpu
from jax.experimental.pallas import tpu_sc as plsc

NC = 2
NS = 16
NW = NC * NS
L = 16

SUB = 128


@functools.lru_cache(maxsize=None)
def _make(V, D, B):
    assert B % (NW * SUB) == 0 and D % L == 0
    bpw = B // NW
    nsub = bpw // SUB
    scale = float(math.sqrt(D))
    mesh = plsc.VectorSubcoreMesh(core_axis_name="c", subcore_axis_name="s")

    @functools.partial(
        pl.kernel,
        mesh=mesh,
        compiler_params=pltpu.CompilerParams(use_tc_tiling_on_sc=True),
        out_type=jax.ShapeDtypeStruct((B, D), jnp.float32),
        scratch_types=[
            pltpu.VMEM((nsub, SUB), jnp.int32),
            pltpu.VMEM((nsub, SUB), jnp.int32),
            pltpu.VMEM((SUB, 2 * D), jnp.float32),
            pltpu.VMEM((SUB, 2 * D), jnp.float32),
            pltpu.VMEM((SUB, D), jnp.float32),
            pltpu.VMEM((SUB, D), jnp.float32),
            pltpu.SemaphoreType.DMA,
            pltpu.SemaphoreType.DMA,
            pltpu.SemaphoreType.DMA,
            pltpu.SemaphoreType.DMA,
        ],
    )
    def emb(table_hbm, ids_hbm, out_hbm,
            idx_v, idx_u, gb0, gb1, sb0, sb1, gsem0, gsem1, ssem0, ssem1):
        wid = lax.axis_index("s") * NC + lax.axis_index("c")
        row0 = wid * bpw
        gbufs, sbufs = (gb0, gb1), (sb0, sb1)
        gsems, ssems = (gsem0, gsem1), (ssem0, ssem1)

        pltpu.sync_copy(ids_hbm.at[wid], idx_v)

        def mk_u(s, carry):
            for j in range(SUB // L):
                idx_u[s, pl.ds(j * L, L)] = lax.shift_right_logical(
                    idx_v[s, pl.ds(j * L, L)], 1)
            return carry
        lax.fori_loop(0, nsub, mk_u, 0)

        def fire_gather(g, b):
            pltpu.async_copy(table_hbm.at[idx_u.at[g]], gbufs[b], gsems[b])

        fire_gather(0, 0)
        fire_gather(1, 1)

        def chunk(g, b):
            pltpu.make_async_copy(
                table_hbm.at[pl.ds(0, SUB)], gbufs[b], gsems[b]).wait()

            @pl.when(g >= 2)
            def _():
                pltpu.make_async_copy(
                    sbufs[b], out_hbm.at[pl.ds(0, SUB)], ssems[b]).wait()

            def srow(r0, carry):
                hv = (idx_v[g, pl.ds(r0 * L, L)] & 1).astype(jnp.float32)
                for rr in range(L):
                    r = r0 * L + rr
                    hf = lax.broadcast(hv[rr], (L,))
                    for j in range(D // L):
                        lo = gbufs[b][r, pl.ds(j * L, L)]
                        hi = gbufs[b][r, pl.ds(D + j * L, L)]
                        sbufs[b][r, pl.ds(j * L, L)] = (
                            (lo + hf * (hi - lo)) * scale)
                return carry
            lax.fori_loop(0, SUB // L, srow, 0)

            @pl.when(g + 2 < nsub)
            def _():
                fire_gather(g + 2, b)

            pltpu.async_copy(
                sbufs[b], out_hbm.at[pl.ds(row0 + g * SUB, SUB)], ssems[b])

        def outer(i, carry):
            chunk(2 * i, 0)
            chunk(2 * i + 1, 1)
            return carry
        lax.fori_loop(0, nsub // 2, outer, 0)

        for b in range(2):
            pltpu.make_async_copy(
                sbufs[b], out_hbm.at[pl.ds(0, SUB)], ssems[b]).wait()

    return emb


def kernel(input_ids, table):
    V, D = table.shape
    B = input_ids.size
    bpw = B // NW
    table2 = table.reshape(V // 2, 2 * D)
    ids3 = input_ids.astype(jnp.int32).reshape(NW, bpw // SUB, SUB)
    out = _make(V, D, B)(table2, ids3)
    return out.reshape(*input_ids.shape, D)


# speedup vs baseline: 1.2344x; 1.0806x over previous
import functools
import math

import jax
import jax.numpy as jnp
from jax import lax
from jax.experimental import pallas as pl
from jax.experimental.pallas import tpu as pltpu
from jax.experimental.pallas import tpu_sc as plsc

NC = 2
NS = 16
NW = NC * NS
L = 16

SUB = 128
TBLK = 2048


@functools.lru_cache(maxsize=None)
def _make_pack(V, D):
    scale = float(math.sqrt(D))

    def body(tt_ref, out_ref):
        t = jnp.transpose(tt_ref[...], (1, 0)) * scale
        out_ref[...] = jnp.concatenate([t, t], axis=1)

    return pl.pallas_call(
        body,
        grid=((V + TBLK - 1) // TBLK,),
        in_specs=[pl.BlockSpec((D, TBLK), lambda i: (0, i))],
        out_specs=pl.BlockSpec((TBLK, 2 * D), lambda i: (i, 0)),
        out_shape=jax.ShapeDtypeStruct((V, 2 * D), jnp.float32),
    )


@functools.lru_cache(maxsize=None)
def _make_gather(V, D, B):
    assert B % (NW * SUB) == 0
    bpw = B // NW
    nsub = bpw // SUB
    mesh = plsc.VectorSubcoreMesh(core_axis_name="c", subcore_axis_name="s")

    @functools.partial(
        pl.kernel,
        mesh=mesh,
        compiler_params=pltpu.CompilerParams(use_tc_tiling_on_sc=True),
        out_type=jax.ShapeDtypeStruct((B, D), jnp.float32),
        scratch_types=[
            pltpu.VMEM((nsub, SUB), jnp.int32),
            pltpu.VMEM((SUB, 2 * D), jnp.float32),
            pltpu.VMEM((SUB, 2 * D), jnp.float32),
            pltpu.VMEM((SUB, D), jnp.float32),
            pltpu.VMEM((SUB, D), jnp.float32),
            pltpu.SemaphoreType.DMA,
            pltpu.SemaphoreType.DMA,
            pltpu.SemaphoreType.DMA,
            pltpu.SemaphoreType.DMA,
        ],
    )
    def gat(tab_hbm, ids_hbm, out_hbm,
            idx_v, gb0, gb1, sb0, sb1, gsem0, gsem1, ssem0, ssem1):
        wid = lax.axis_index("s") * NC + lax.axis_index("c")
        row0 = wid * bpw
        gbufs, sbufs = (gb0, gb1), (sb0, sb1)
        gsems, ssems = (gsem0, gsem1), (ssem0, ssem1)

        pltpu.sync_copy(ids_hbm.at[wid], idx_v)

        def fire_gather(g, b):
            pltpu.async_copy(tab_hbm.at[idx_v.at[g]], gbufs[b], gsems[b])

        fire_gather(0, 0)
        fire_gather(1, 1)

        def chunk(g, b):
            pltpu.make_async_copy(
                tab_hbm.at[pl.ds(0, SUB)], gbufs[b], gsems[b]).wait()

            @pl.when(g >= 2)
            def _():
                pltpu.make_async_copy(
                    sbufs[b], out_hbm.at[pl.ds(0, SUB)], ssems[b]).wait()

            def srow(r4, carry):
                for rr in range(4):
                    r = r4 * 4 + rr
                    for j in range(D // L):
                        sbufs[b][r, pl.ds(j * L, L)] = (
                            gbufs[b][r, pl.ds(j * L, L)])
                return carry
            lax.fori_loop(0, SUB // 4, srow, 0)

            @pl.when(g + 2 < nsub)
            def _():
                fire_gather(g + 2, b)

            pltpu.async_copy(
                sbufs[b], out_hbm.at[pl.ds(row0 + g * SUB, SUB)], ssems[b])

        def outer(i, carry):
            chunk(2 * i, 0)
            chunk(2 * i + 1, 1)
            return carry
        lax.fori_loop(0, nsub // 2, outer, 0)

        for b in range(2):
            pltpu.make_async_copy(
                sbufs[b], out_hbm.at[pl.ds(0, SUB)], ssems[b]).wait()

    return gat


def kernel(input_ids, table):
    V, D = table.shape
    B = input_ids.size
    bpw = B // NW
    packed = _make_pack(V, D)(table.T)
    ids3 = input_ids.astype(jnp.int32).reshape(NW, bpw // SUB, SUB)
    out = _make_gather(V, D, B)(packed, ids3)
    return out.reshape(*input_ids.shape, D)


# speedup vs baseline: 1.6444x; 1.3322x over previous
import functools
import math

import jax
import jax.numpy as jnp
from jax import lax
from jax.experimental import pallas as pl
from jax.experimental.pallas import tpu as pltpu
from jax.experimental.pallas import tpu_sc as plsc

NC = 2
NS = 16
NW = NC * NS
L = 16

SUB = 128
TBLK = 8192


@functools.lru_cache(maxsize=None)
def _make_pack(V, D):
    scale = float(math.sqrt(D))

    def body(tt_ref, out_ref):
        out_ref[:, 0:D] = jnp.transpose(tt_ref[...], (1, 0)) * scale

    return pl.pallas_call(
        body,
        grid=((V + TBLK - 1) // TBLK,),
        in_specs=[pl.BlockSpec((D, TBLK), lambda i: (0, i))],
        out_specs=pl.BlockSpec((TBLK, 2 * D), lambda i: (i, 0)),
        out_shape=jax.ShapeDtypeStruct((V, 2 * D), jnp.float32),
    )


@functools.lru_cache(maxsize=None)
def _make_gather(V, D, B):
    assert B % (NW * SUB) == 0
    bpw = B // NW
    nsub = bpw // SUB
    mesh = plsc.VectorSubcoreMesh(core_axis_name="c", subcore_axis_name="s")

    @functools.partial(
        pl.kernel,
        mesh=mesh,
        compiler_params=pltpu.CompilerParams(use_tc_tiling_on_sc=True),
        out_type=jax.ShapeDtypeStruct((B, D), jnp.float32),
        scratch_types=[
            pltpu.VMEM((nsub, SUB), jnp.int32),
            pltpu.VMEM((SUB, 2 * D), jnp.float32),
            pltpu.VMEM((SUB, 2 * D), jnp.float32),
            pltpu.VMEM((SUB, D), jnp.float32),
            pltpu.VMEM((SUB, D), jnp.float32),
            pltpu.SemaphoreType.DMA,
            pltpu.SemaphoreType.DMA,
            pltpu.SemaphoreType.DMA,
            pltpu.SemaphoreType.DMA,
        ],
    )
    def gat(tab_hbm, ids_hbm, out_hbm,
            idx_v, gb0, gb1, sb0, sb1, gsem0, gsem1, ssem0, ssem1):
        wid = lax.axis_index("s") * NC + lax.axis_index("c")
        row0 = wid * bpw
        gbufs, sbufs = (gb0, gb1), (sb0, sb1)
        gsems, ssems = (gsem0, gsem1), (ssem0, ssem1)

        pltpu.sync_copy(ids_hbm.at[wid], idx_v)

        def fire_gather(g, b):
            pltpu.async_copy(tab_hbm.at[idx_v.at[g]], gbufs[b], gsems[b])

        fire_gather(0, 0)
        fire_gather(1, 1)

        def chunk(g, b):
            pltpu.make_async_copy(
                tab_hbm.at[pl.ds(0, SUB)], gbufs[b], gsems[b]).wait()

            @pl.when(g >= 2)
            def _():
                pltpu.make_async_copy(
                    sbufs[b], out_hbm.at[pl.ds(0, SUB)], ssems[b]).wait()

            def srow(r4, carry):
                for rr in range(4):
                    r = r4 * 4 + rr
                    for j in range(D // L):
                        sbufs[b][r, pl.ds(j * L, L)] = (
                            gbufs[b][r, pl.ds(j * L, L)])
                return carry
            lax.fori_loop(0, SUB // 4, srow, 0)

            @pl.when(g + 2 < nsub)
            def _():
                fire_gather(g + 2, b)

            pltpu.async_copy(
                sbufs[b], out_hbm.at[pl.ds(row0 + g * SUB, SUB)], ssems[b])

        def outer(i, carry):
            chunk(2 * i, 0)
            chunk(2 * i + 1, 1)
            return carry
        lax.fori_loop(0, nsub // 2, outer, 0)

        for b in range(2):
            pltpu.make_async_copy(
                sbufs[b], out_hbm.at[pl.ds(0, SUB)], ssems[b]).wait()

    return gat


def kernel(input_ids, table):
    V, D = table.shape
    B = input_ids.size
    bpw = B // NW
    packed = _make_pack(V, D)(table.T)
    ids3 = input_ids.astype(jnp.int32).reshape(NW, bpw // SUB, SUB)
    out = _make_gather(V, D, B)(packed, ids3)
    return out.reshape(*input_ids.shape, D)


# speedup vs baseline: 1.6892x; 1.0273x over previous
import functools
import math

import jax
import jax.numpy as jnp
from jax import lax
from jax.experimental import pallas as pl
from jax.experimental.pallas import tpu as pltpu
from jax.experimental.pallas import tpu_sc as plsc

NC = 2
NS = 16
NW = NC * NS
L = 16

SUB = 128
TBLK = 16384


@functools.lru_cache(maxsize=None)
def _make_pack(V, D):
    scale = float(math.sqrt(D))

    def body(tt_ref, out_ref):
        out_ref[:, 0:D] = jnp.transpose(tt_ref[...], (1, 0)) * scale

    return pl.pallas_call(
        body,
        grid=((V + TBLK - 1) // TBLK,),
        in_specs=[pl.BlockSpec((D, TBLK), lambda i: (0, i))],
        out_specs=pl.BlockSpec((TBLK, 2 * D), lambda i: (i, 0)),
        out_shape=jax.ShapeDtypeStruct((V, 2 * D), jnp.float32),
    )


@functools.lru_cache(maxsize=None)
def _make_gather(V, D, B):
    assert B % (NW * SUB) == 0
    bpw = B // NW
    nsub = bpw // SUB
    mesh = plsc.VectorSubcoreMesh(core_axis_name="c", subcore_axis_name="s")

    @functools.partial(
        pl.kernel,
        mesh=mesh,
        compiler_params=pltpu.CompilerParams(use_tc_tiling_on_sc=True),
        out_type=jax.ShapeDtypeStruct((B, D), jnp.float32),
        scratch_types=[
            pltpu.VMEM((nsub, SUB), jnp.int32),
            pltpu.VMEM((SUB, 2 * D), jnp.float32),
            pltpu.VMEM((SUB, 2 * D), jnp.float32),
            pltpu.VMEM((SUB, D), jnp.float32),
            pltpu.VMEM((SUB, D), jnp.float32),
            pltpu.SemaphoreType.DMA,
            pltpu.SemaphoreType.DMA,
            pltpu.SemaphoreType.DMA,
            pltpu.SemaphoreType.DMA,
        ],
    )
    def gat(tab_hbm, ids_hbm, out_hbm,
            idx_v, gb0, gb1, sb0, sb1, gsem0, gsem1, ssem0, ssem1):
        wid = lax.axis_index("s") * NC + lax.axis_index("c")
        row0 = wid * bpw
        gbufs, sbufs = (gb0, gb1), (sb0, sb1)
        gsems, ssems = (gsem0, gsem1), (ssem0, ssem1)

        pltpu.sync_copy(ids_hbm.at[wid], idx_v)

        def fire_gather(g, b):
            pltpu.async_copy(tab_hbm.at[idx_v.at[g]], gbufs[b], gsems[b])

        fire_gather(0, 0)
        fire_gather(1, 1)

        def chunk(g, b):
            pltpu.make_async_copy(
                tab_hbm.at[pl.ds(0, SUB)], gbufs[b], gsems[b]).wait()

            @pl.when(g >= 2)
            def _():
                pltpu.make_async_copy(
                    sbufs[b], out_hbm.at[pl.ds(0, SUB)], ssems[b]).wait()

            def srow(r4, carry):
                for rr in range(4):
                    r = r4 * 4 + rr
                    for j in range(D // L):
                        sbufs[b][r, pl.ds(j * L, L)] = (
                            gbufs[b][r, pl.ds(j * L, L)])
                return carry
            lax.fori_loop(0, SUB // 4, srow, 0)

            @pl.when(g + 2 < nsub)
            def _():
                fire_gather(g + 2, b)

            pltpu.async_copy(
                sbufs[b], out_hbm.at[pl.ds(row0 + g * SUB, SUB)], ssems[b])

        def outer(i, carry):
            chunk(2 * i, 0)
            chunk(2 * i + 1, 1)
            return carry
        lax.fori_loop(0, nsub // 2, outer, 0)

        for b in range(2):
            pltpu.make_async_copy(
                sbufs[b], out_hbm.at[pl.ds(0, SUB)], ssems[b]).wait()

    return gat


def kernel(input_ids, table):
    V, D = table.shape
    B = input_ids.size
    bpw = B // NW
    packed = _make_pack(V, D)(table.T)
    ids3 = input_ids.astype(jnp.int32).reshape(NW, bpw // SUB, SUB)
    out = _make_gather(V, D, B)(packed, ids3)
    return out.reshape(*input_ids.shape, D)


# speedup vs baseline: 1.7016x; 1.0073x over previous
import functools
import math

import jax
import jax.numpy as jnp
from jax import lax
from jax.experimental import pallas as pl
from jax.experimental.pallas import tpu as pltpu
from jax.experimental.pallas import tpu_sc as plsc

NC = 2
NS = 16
NW = NC * NS
L = 16

SUB = 128
TBLK = 32768


@functools.lru_cache(maxsize=None)
def _make_pack(V, D):
    scale = float(math.sqrt(D))

    def body(tt_ref, out_ref):
        out_ref[:, 0:D] = jnp.transpose(tt_ref[...], (1, 0)) * scale

    return pl.pallas_call(
        body,
        grid=((V + TBLK - 1) // TBLK,),
        in_specs=[pl.BlockSpec((D, TBLK), lambda i: (0, i))],
        out_specs=pl.BlockSpec((TBLK, 2 * D), lambda i: (i, 0)),
        out_shape=jax.ShapeDtypeStruct((V, 2 * D), jnp.float32),
    )


@functools.lru_cache(maxsize=None)
def _make_gather(V, D, B):
    assert B % (NW * SUB) == 0
    bpw = B // NW
    nsub = bpw // SUB
    mesh = plsc.VectorSubcoreMesh(core_axis_name="c", subcore_axis_name="s")

    @functools.partial(
        pl.kernel,
        mesh=mesh,
        compiler_params=pltpu.CompilerParams(use_tc_tiling_on_sc=True),
        out_type=jax.ShapeDtypeStruct((B, D), jnp.float32),
        scratch_types=[
            pltpu.VMEM((nsub, SUB), jnp.int32),
            pltpu.VMEM((SUB, 2 * D), jnp.float32),
            pltpu.VMEM((SUB, 2 * D), jnp.float32),
            pltpu.VMEM((SUB, D), jnp.float32),
            pltpu.VMEM((SUB, D), jnp.float32),
            pltpu.SemaphoreType.DMA,
            pltpu.SemaphoreType.DMA,
            pltpu.SemaphoreType.DMA,
            pltpu.SemaphoreType.DMA,
        ],
    )
    def gat(tab_hbm, ids_hbm, out_hbm,
            idx_v, gb0, gb1, sb0, sb1, gsem0, gsem1, ssem0, ssem1):
        wid = lax.axis_index("s") * NC + lax.axis_index("c")
        row0 = wid * bpw
        gbufs, sbufs = (gb0, gb1), (sb0, sb1)
        gsems, ssems = (gsem0, gsem1), (ssem0, ssem1)

        pltpu.sync_copy(ids_hbm.at[wid], idx_v)

        def fire_gather(g, b):
            pltpu.async_copy(tab_hbm.at[idx_v.at[g]], gbufs[b], gsems[b])

        fire_gather(0, 0)
        fire_gather(1, 1)

        def chunk(g, b):
            pltpu.make_async_copy(
                tab_hbm.at[pl.ds(0, SUB)], gbufs[b], gsems[b]).wait()

            @pl.when(g >= 2)
            def _():
                pltpu.make_async_copy(
                    sbufs[b], out_hbm.at[pl.ds(0, SUB)], ssems[b]).wait()

            def srow(r4, carry):
                for rr in range(4):
                    r = r4 * 4 + rr
                    for j in range(D // L):
                        sbufs[b][r, pl.ds(j * L, L)] = (
                            gbufs[b][r, pl.ds(j * L, L)])
                return carry
            lax.fori_loop(0, SUB // 4, srow, 0)

            @pl.when(g + 2 < nsub)
            def _():
                fire_gather(g + 2, b)

            pltpu.async_copy(
                sbufs[b], out_hbm.at[pl.ds(row0 + g * SUB, SUB)], ssems[b])

        def outer(i, carry):
            chunk(2 * i, 0)
            chunk(2 * i + 1, 1)
            return carry
        lax.fori_loop(0, nsub // 2, outer, 0)

        for b in range(2):
            pltpu.make_async_copy(
                sbufs[b], out_hbm.at[pl.ds(0, SUB)], ssems[b]).wait()

    return gat


def kernel(input_ids, table):
    V, D = table.shape
    B = input_ids.size
    bpw = B // NW
    packed = _make_pack(V, D)(table.T)
    ids3 = input_ids.astype(jnp.int32).reshape(NW, bpw // SUB, SUB)
    out = _make_gather(V, D, B)(packed, ids3)
    return out.reshape(*input_ids.shape, D)
